# baseline (speedup 1.0000x reference)
import jax
import jax.numpy as jnp
from jax.experimental import pallas as pl
from jax.experimental.pallas import tpu as pltpu

_N_GENDER = 2
_N_AGE = 7


def _round_up(x, m):
    return (x + m - 1) // m * m


def _fused_gather_kernel(x_ref, w_ref, out_ref):
    k_pad = w_ref.shape[0]
    tile_b = out_ref.shape[0]

    kcol = jax.lax.broadcasted_iota(jnp.int32, (tile_b, k_pad), 1)
    g = x_ref[:, 0:1]
    a = x_ref[:, 1:2] + _N_GENDER
    o = x_ref[:, 2:3] + (_N_GENDER + _N_AGE)
    onehot = ((kcol == g) | (kcol == a) | (kcol == o)).astype(jnp.float32)

    out_ref[...] = jnp.dot(onehot, w_ref[...],
                           preferred_element_type=jnp.float32)


def kernel(x1, w_blk, *, tile_b=8192):
    B = x1.shape[0]
    assert x1.shape[1] == 3
    k_pad, out_dim = w_blk.shape
    x1 = x1.astype(jnp.int32)

    if B <= 256:
        tile_b = B
    else:
        tile_b = max(256, min(int(tile_b), 16384))
        tile_b = min(tile_b, _round_up(pl.cdiv(B, 2), 256))
        tile_b = _round_up(tile_b, 256)
    grid = (pl.cdiv(B, tile_b),)

    return pl.pallas_call(
        _fused_gather_kernel,
        out_shape=jax.ShapeDtypeStruct((B, out_dim), jnp.float32),
        grid=grid,
        in_specs=[
            pl.BlockSpec((tile_b, 3), lambda i: (i, 0)),
            pl.BlockSpec((k_pad, out_dim), lambda i: (0, 0)),
        ],
        out_specs=pl.BlockSpec((tile_b, out_dim), lambda i: (i, 0)),
        compiler_params=pltpu.CompilerParams(
            dimension_semantics=("parallel",)),
    )(x1, w_blk)


# speedup vs baseline: 2.0540x; 2.0540x over previous
import jax
import jax.numpy as jnp
from jax.experimental import pallas as pl
from jax.experimental.pallas import tpu as pltpu

_N_GENDER = 2
_N_AGE = 7
_OUT_DIM = 96
_PAD_DIM = 128


def _round_up(x, m):
    return (x + m - 1) // m * m


def _fused_gather_kernel(idx_ref, w_ref, out_ref):
    k_pad = w_ref.shape[0]
    tile_b = out_ref.shape[0]

    krow = jax.lax.broadcasted_iota(jnp.int32, (k_pad, tile_b), 0)
    g = idx_ref[0:1, :]
    a = idx_ref[1:2, :] + _N_GENDER
    o = idx_ref[2:3, :] + (_N_GENDER + _N_AGE)
    onehot_t = ((krow == g) | (krow == a) | (krow == o)).astype(jnp.float32)

    onehot = jnp.transpose(onehot_t)
    out_ref[...] = jnp.dot(onehot, w_ref[...],
                           preferred_element_type=jnp.float32)


def kernel(x1, w_blk, *, tile_b=8192):
    B = x1.shape[0]
    assert x1.shape[1] == 3
    k_pad, out_dim = w_blk.shape

    x1_t = jnp.transpose(x1.astype(jnp.int32))

    w128 = jnp.zeros((k_pad, _PAD_DIM), jnp.float32).at[:, :out_dim].set(w_blk)

    if B <= 128:
        tile_b = B
    else:
        tile_b = max(128, min(int(tile_b), 32768))
        tile_b = min(tile_b, _round_up(pl.cdiv(B, 2), 128))
        tile_b = _round_up(tile_b, 128)
    grid = (pl.cdiv(B, tile_b),)

    out128 = pl.pallas_call(
        _fused_gather_kernel,
        out_shape=jax.ShapeDtypeStruct((B, _PAD_DIM), jnp.float32),
        grid=grid,
        in_specs=[
            pl.BlockSpec((3, tile_b), lambda i: (0, i)),
            pl.BlockSpec((k_pad, _PAD_DIM), lambda i: (0, 0)),
        ],
        out_specs=pl.BlockSpec((tile_b, _PAD_DIM), lambda i: (i, 0)),
        compiler_params=pltpu.CompilerParams(
            dimension_semantics=("parallel",)),
    )(x1_t, w128)

    return jax.lax.slice(out128, (0, 0), (B, out_dim))


# speedup vs baseline: 6.6855x; 3.2548x over previous
import jax
import jax.numpy as jnp
from jax.experimental import pallas as pl
from jax.experimental.pallas import tpu as pltpu

_N_GENDER = 2
_N_AGE = 7


def _round_up(x, m):
    return (x + m - 1) // m * m


def _fused_gather_kernel_t(idx_ref, wt_ref, out_ref):
    k_pad = wt_ref.shape[1]
    tile_b = out_ref.shape[1]

    krow = jax.lax.broadcasted_iota(jnp.int32, (k_pad, tile_b), 0)
    g = idx_ref[0:1, :]
    a = idx_ref[1:2, :] + _N_GENDER
    o = idx_ref[2:3, :] + (_N_GENDER + _N_AGE)
    onehot_t = ((krow == g) | (krow == a) | (krow == o)).astype(jnp.float32)

    out_ref[...] = jnp.dot(wt_ref[...], onehot_t,
                           preferred_element_type=jnp.float32)


def kernel(x1, w_blk, *, tile_b=8192):
    B = x1.shape[0]
    assert x1.shape[1] == 3
    k_pad, out_dim = w_blk.shape

    x1_t = jnp.transpose(x1.astype(jnp.int32))
    w_t = jnp.transpose(w_blk)

    if B <= 128:
        tile_b = B
    else:
        tile_b = max(128, min(int(tile_b), 32768))
        tile_b = min(tile_b, _round_up(pl.cdiv(B, 2), 128))
        tile_b = _round_up(tile_b, 128)
    grid = (pl.cdiv(B, tile_b),)

    out_t = pl.pallas_call(
        _fused_gather_kernel_t,
        out_shape=jax.ShapeDtypeStruct((out_dim, B), jnp.float32),
        grid=grid,
        in_specs=[
            pl.BlockSpec((3, tile_b), lambda i: (0, i)),
            pl.BlockSpec((out_dim, k_pad), lambda i: (0, 0)),
        ],
        out_specs=pl.BlockSpec((out_dim, tile_b), lambda i: (0, i)),
        compiler_params=pltpu.CompilerParams(
            dimension_semantics=("parallel",)),
    )(x1_t, w_t)

    return jnp.transpose(out_t)


# speedup vs baseline: 8.5484x; 1.2786x over previous
import jax
import jax.numpy as jnp
from jax.experimental import pallas as pl
from jax.experimental.pallas import tpu as pltpu

_N_GENDER = 2
_N_AGE = 7


def _round_up(x, m):
    return (x + m - 1) // m * m


def _fused_gather_kernel_t(idx_ref, wt_ref, out_ref):
    k_pad = wt_ref.shape[1]
    tile_b = out_ref.shape[1]

    krow = jax.lax.broadcasted_iota(jnp.int32, (k_pad, tile_b), 0)
    g = idx_ref[0:1, :]
    a = idx_ref[1:2, :] + _N_GENDER
    o = idx_ref[2:3, :] + (_N_GENDER + _N_AGE)
    onehot_t = ((krow == g) | (krow == a) | (krow == o)).astype(jnp.float32)

    out_ref[...] = jnp.dot(wt_ref[...], onehot_t,
                           preferred_element_type=jnp.float32)


def kernel(x1, w_blk, *, tile_b=16384):
    B = x1.shape[0]
    assert x1.shape[1] == 3
    k_pad, out_dim = w_blk.shape

    x1_t = jnp.transpose(x1.astype(jnp.int32))
    w_t = jnp.transpose(w_blk)

    if B <= 128:
        tile_b = B
    else:
        tile_b = max(128, min(int(tile_b), 32768))
        tile_b = min(tile_b, _round_up(pl.cdiv(B, 2), 128))
        tile_b = _round_up(tile_b, 128)
    grid = (pl.cdiv(B, tile_b),)

    out_t = pl.pallas_call(
        _fused_gather_kernel_t,
        out_shape=jax.ShapeDtypeStruct((out_dim, B), jnp.float32),
        grid=grid,
        in_specs=[
            pl.BlockSpec((3, tile_b), lambda i: (0, i)),
            pl.BlockSpec((out_dim, k_pad), lambda i: (0, 0)),
        ],
        out_specs=pl.BlockSpec((out_dim, tile_b), lambda i: (0, i)),
        compiler_params=pltpu.CompilerParams(
            dimension_semantics=("parallel",)),
    )(x1_t, w_t)

    return jnp.transpose(out_t)


# speedup vs baseline: 8.7702x; 1.0259x over previous
import jax
import jax.numpy as jnp
from jax.experimental import pallas as pl
from jax.experimental.pallas import tpu as pltpu

_N_GENDER = 2
_N_AGE = 7


def _round_up(x, m):
    return (x + m - 1) // m * m


def _fused_gather_kernel_t(idx_ref, wt_ref, out_ref):
    k_pad = wt_ref.shape[1]
    tile_b = out_ref.shape[1]

    krow = jax.lax.broadcasted_iota(jnp.int32, (k_pad, tile_b), 0)
    g = idx_ref[0:1, :]
    a = idx_ref[1:2, :] + _N_GENDER
    o = idx_ref[2:3, :] + (_N_GENDER + _N_AGE)
    onehot_t = ((krow == g) | (krow == a) | (krow == o)).astype(jnp.float32)

    out_ref[...] = jnp.dot(wt_ref[...], onehot_t,
                           preferred_element_type=jnp.float32)


def kernel(x1, w_blk, *, tile_b=32768):
    B = x1.shape[0]
    assert x1.shape[1] == 3
    k_pad, out_dim = w_blk.shape

    x1_t = jnp.transpose(x1.astype(jnp.int32))
    w_t = jnp.transpose(w_blk)

    if B <= 128:
        tile_b = B
    else:
        tile_b = max(128, min(int(tile_b), 32768))
        tile_b = min(tile_b, _round_up(pl.cdiv(B, 2), 128))
        tile_b = _round_up(tile_b, 128)
    grid = (pl.cdiv(B, tile_b),)

    out_t = pl.pallas_call(
        _fused_gather_kernel_t,
        out_shape=jax.ShapeDtypeStruct((out_dim, B), jnp.float32),
        grid=grid,
        in_specs=[
            pl.BlockSpec((3, tile_b), lambda i: (0, i)),
            pl.BlockSpec((out_dim, k_pad), lambda i: (0, 0)),
        ],
        out_specs=pl.BlockSpec((out_dim, tile_b), lambda i: (0, i)),
        compiler_params=pltpu.CompilerParams(
            dimension_semantics=("parallel",)),
    )(x1_t, w_t)

    return jnp.transpose(out_t)


# speedup vs baseline: 8.7924x; 1.0025x over previous
import jax
import jax.numpy as jnp
from jax.experimental import pallas as pl
from jax.experimental.pallas import tpu as pltpu

_N_GENDER = 2
_N_AGE = 7


def _round_up(x, m):
    return (x + m - 1) // m * m


def _fused_gather_kernel_t(idx_ref, wt_ref, out_ref):
    k_pad = wt_ref.shape[1]
    tile_b = out_ref.shape[1]

    krow = jax.lax.broadcasted_iota(jnp.int32, (k_pad, tile_b), 0)
    g = idx_ref[0:1, :]
    a = idx_ref[1:2, :] + _N_GENDER
    o = idx_ref[2:3, :] + (_N_GENDER + _N_AGE)
    onehot_t = ((krow == g) | (krow == a) | (krow == o)).astype(jnp.float32)

    out_ref[...] = jnp.dot(wt_ref[...], onehot_t,
                           preferred_element_type=jnp.float32)


def kernel(x1, w_blk, *, tile_b=65536):
    B = x1.shape[0]
    assert x1.shape[1] == 3
    k_pad, out_dim = w_blk.shape

    x1_t = jnp.transpose(x1.astype(jnp.int32))
    w_t = jnp.transpose(w_blk)

    if B <= 128:
        tile_b = B
    else:
        tile_b = max(128, min(int(tile_b), 65536))
        tile_b = min(tile_b, _round_up(pl.cdiv(B, 2), 128))
        tile_b = _round_up(tile_b, 128)
    grid = (pl.cdiv(B, tile_b),)

    out_t = pl.pallas_call(
        _fused_gather_kernel_t,
        out_shape=jax.ShapeDtypeStruct((out_dim, B), jnp.float32),
        grid=grid,
        in_specs=[
            pl.BlockSpec((3, tile_b), lambda i: (0, i)),
            pl.BlockSpec((out_dim, k_pad), lambda i: (0, 0)),
        ],
        out_specs=pl.BlockSpec((out_dim, tile_b), lambda i: (0, i)),
        compiler_params=pltpu.CompilerParams(
            dimension_semantics=("parallel",)),
    )(x1_t, w_t)

    return jnp.transpose(out_t)
